# baseline (speedup 1.0000x reference)
import functools

import jax
import jax.numpy as jnp
from jax import lax
from jax.experimental import pallas as pl
from jax.experimental.pallas import tpu as pltpu
from jax.experimental.pallas import tpu_sc as plsc

_NEG_INF = float("-inf")
_LOG2E = 1.4426950408889634


def _reduce_body(V, C, NBLK, x_ref, logz_ref, mode_ref, m_sc, s_sc, bv_sc, bi_sc):
    j = pl.program_id(0)
    B = x_ref.shape[0]
    col = lax.broadcasted_iota(jnp.int32, (B, C), 1)

    def _stats(masked):
        x = x_ref[...]
        if masked:
            x = jnp.where(j * C + col < V, x, _NEG_INF)
        bmax = jnp.max(x, axis=-1, keepdims=True)
        bidx = jnp.min(jnp.where(x == bmax, col, 2**31 - 1), axis=-1,
                       keepdims=True) + j * C
        bsum = jnp.sum(jnp.exp2(x * _LOG2E + bmax * (-_LOG2E)), axis=-1,
                       keepdims=True)
        return bmax, bidx, bsum

    def _acc(bmax, bidx, bsum):
        m_old = m_sc[...]
        m_new = jnp.maximum(m_old, bmax)
        s_sc[...] = s_sc[...] * jnp.exp(m_old - m_new) + bsum * jnp.exp(bmax - m_new)
        m_sc[...] = m_new
        better = bmax > bv_sc[...]
        bv_sc[...] = jnp.where(better, bmax, bv_sc[...])
        bi_sc[...] = jnp.where(better, bidx, bi_sc[...])

    tail_masked = V % C != 0

    @pl.when(j == 0)
    def _init():
        bmax, bidx, bsum = _stats(masked=tail_masked and NBLK == 1)
        m_sc[...] = bmax
        s_sc[...] = bsum
        bv_sc[...] = bmax
        bi_sc[...] = bidx

    @pl.when((j > 0) & (j < NBLK - 1))
    def _mid():
        _acc(*_stats(masked=False))

    @pl.when((j == NBLK - 1) & (j > 0))
    def _last():
        _acc(*_stats(masked=tail_masked))

    @pl.when(j == NBLK - 1)
    def _fin():
        logz_ref[...] = m_sc[...] + jnp.log(s_sc[...])
        mode_ref[...] = bi_sc[...]


def _fused_pass(logits, C=32768):
    B, V = logits.shape
    NBLK = pl.cdiv(V, C)
    return pl.pallas_call(
        functools.partial(_reduce_body, V, C, NBLK),
        grid=(NBLK,),
        in_specs=[pl.BlockSpec((B, C), lambda j: (0, j))],
        out_specs=[pl.BlockSpec((B, 1), lambda j: (0, 0)),
                   pl.BlockSpec((B, 1), lambda j: (0, 0))],
        out_shape=[jax.ShapeDtypeStruct((B, 1), jnp.float32),
                   jax.ShapeDtypeStruct((B, 1), jnp.int32)],
        scratch_shapes=[pltpu.VMEM((B, 1), jnp.float32),
                        pltpu.VMEM((B, 1), jnp.float32),
                        pltpu.VMEM((B, 1), jnp.float32),
                        pltpu.VMEM((B, 1), jnp.int32)],
    )(logits)


def _sc_gather(logits, idx):
    info = plsc.get_sparse_core_info()
    nc, L = info.num_cores, info.num_lanes
    B, V = logits.shape
    nw = B // L
    mesh = plsc.VectorSubcoreMesh(core_axis_name="c", subcore_axis_name="s")

    @functools.partial(
        pl.kernel, mesh=mesh,
        out_type=jax.ShapeDtypeStruct((B, 128), jnp.float32),
        scratch_types=[pltpu.VMEM((L,), jnp.int32),
                       pltpu.VMEM((L, 8, 128), jnp.float32),
                       pltpu.VMEM((L, 128), jnp.float32)],
    )
    def k(x_hbm, idx_hbm, out_hbm, idx_v, buf_v, seg_v):
        wid = lax.axis_index("s") * nc + lax.axis_index("c")

        @pl.when(wid < nw)
        def _():
            row0 = wid * L
            pltpu.sync_copy(idx_hbm.at[pl.ds(row0, L)], idx_v)
            idx_reg = idx_v[...]
            for i in range(L):
                base = pl.multiple_of(lax.bitwise_and(idx_reg[i], -128), 128)
                r0 = pl.multiple_of(row0 + (i & ~7), 8)
                pltpu.sync_copy(x_hbm.at[pl.ds(r0, 8), pl.ds(base, 128)],
                                buf_v.at[i])
                for k16 in range(8):
                    seg_v[i, pl.ds(16 * k16, 16)] = buf_v[i, i & 7, pl.ds(16 * k16, 16)]
            pltpu.sync_copy(
                seg_v, out_hbm.at[pl.ds(pl.multiple_of(row0, 8), L), :])

    return k(logits, idx)


def _combine_body(a_ref, seg_ref, logz_ref, lp_ref):
    B = a_ref.shape[0]
    c = lax.bitwise_and(a_ref[...], 127)
    col = lax.broadcasted_iota(jnp.int32, (B, 128), 1)
    g = jnp.sum(jnp.where(col == c, seg_ref[...], 0.0), axis=-1, keepdims=True)
    lp_ref[...] = g - logz_ref[...]


def _combine(actions, seg, logz):
    B = actions.shape[0]
    return pl.pallas_call(
        _combine_body,
        out_shape=jax.ShapeDtypeStruct((B, 1), jnp.float32),
    )(actions, seg, logz)


def kernel(logits, actions):
    B, V = logits.shape
    a = actions.reshape(B).astype(jnp.int32)
    seg = _sc_gather(logits, a)
    logz, mode = _fused_pass(logits)
    log_probs = _combine(actions.astype(jnp.int32), seg, logz)
    return log_probs, mode


# speedup vs baseline: 1.0380x; 1.0380x over previous
import functools

import jax
import jax.numpy as jnp
from jax import lax
from jax.experimental import pallas as pl
from jax.experimental.pallas import tpu as pltpu
from jax.experimental.pallas import tpu_sc as plsc

_NEG_INF = float("-inf")
_LOG2E = 1.4426950408889634


def _reduce_body(V, C, NBLK, x_ref, logz_ref, mode_ref, m_sc, s_sc, bv_sc, bi_sc):
    j = pl.program_id(0)
    B = x_ref.shape[0]
    col = lax.broadcasted_iota(jnp.int32, (B, C), 1)

    def _stats(masked):
        x = x_ref[...]
        if masked:
            x = jnp.where(j * C + col < V, x, _NEG_INF)
        bmax = jnp.max(x, axis=-1, keepdims=True)
        bidx = jnp.min(jnp.where(x == bmax, col, 2**31 - 1), axis=-1,
                       keepdims=True) + j * C
        bsum = jnp.sum(jnp.exp(x - bmax), axis=-1, keepdims=True)
        return bmax, bidx, bsum

    def _acc(bmax, bidx, bsum):
        m_old = m_sc[...]
        m_new = jnp.maximum(m_old, bmax)
        s_sc[...] = s_sc[...] * jnp.exp(m_old - m_new) + bsum * jnp.exp(bmax - m_new)
        m_sc[...] = m_new
        better = bmax > bv_sc[...]
        bv_sc[...] = jnp.where(better, bmax, bv_sc[...])
        bi_sc[...] = jnp.where(better, bidx, bi_sc[...])

    tail_masked = V % C != 0

    @pl.when(j == 0)
    def _init():
        bmax, bidx, bsum = _stats(masked=tail_masked and NBLK == 1)
        m_sc[...] = bmax
        s_sc[...] = bsum
        bv_sc[...] = bmax
        bi_sc[...] = bidx

    @pl.when((j > 0) & (j < NBLK - 1))
    def _mid():
        _acc(*_stats(masked=False))

    @pl.when((j == NBLK - 1) & (j > 0))
    def _last():
        _acc(*_stats(masked=tail_masked))

    @pl.when(j == NBLK - 1)
    def _fin():
        logz_ref[...] = m_sc[...] + jnp.log(s_sc[...])
        mode_ref[...] = bi_sc[...]


def _fused_pass(logits, C=32768):
    B, V = logits.shape
    NBLK = pl.cdiv(V, C)
    return pl.pallas_call(
        functools.partial(_reduce_body, V, C, NBLK),
        grid=(NBLK,),
        in_specs=[pl.BlockSpec((B, C), lambda j: (0, j))],
        out_specs=[pl.BlockSpec((B, 1), lambda j: (0, 0)),
                   pl.BlockSpec((B, 1), lambda j: (0, 0))],
        out_shape=[jax.ShapeDtypeStruct((B, 1), jnp.float32),
                   jax.ShapeDtypeStruct((B, 1), jnp.int32)],
        scratch_shapes=[pltpu.VMEM((B, 1), jnp.float32),
                        pltpu.VMEM((B, 1), jnp.float32),
                        pltpu.VMEM((B, 1), jnp.float32),
                        pltpu.VMEM((B, 1), jnp.int32)],
    )(logits)


def _sc_gather(logits, idx):
    info = plsc.get_sparse_core_info()
    nc, L = info.num_cores, info.num_lanes
    B, V = logits.shape
    nw = B // L
    mesh = plsc.VectorSubcoreMesh(core_axis_name="c", subcore_axis_name="s")

    @functools.partial(
        pl.kernel, mesh=mesh,
        out_type=jax.ShapeDtypeStruct((B, 128), jnp.float32),
        scratch_types=[pltpu.VMEM((L,), jnp.int32),
                       pltpu.VMEM((L, 8, 128), jnp.float32),
                       pltpu.VMEM((L, 128), jnp.float32)],
    )
    def k(x_hbm, idx_hbm, out_hbm, idx_v, buf_v, seg_v):
        wid = lax.axis_index("s") * nc + lax.axis_index("c")

        @pl.when(wid < nw)
        def _():
            row0 = wid * L
            pltpu.sync_copy(idx_hbm.at[pl.ds(row0, L)], idx_v)
            idx_reg = idx_v[...]
            for i in range(L):
                base = pl.multiple_of(lax.bitwise_and(idx_reg[i], -128), 128)
                r0 = pl.multiple_of(row0 + (i & ~7), 8)
                pltpu.sync_copy(x_hbm.at[pl.ds(r0, 8), pl.ds(base, 128)],
                                buf_v.at[i])
                for k16 in range(8):
                    seg_v[i, pl.ds(16 * k16, 16)] = buf_v[i, i & 7, pl.ds(16 * k16, 16)]
            pltpu.sync_copy(
                seg_v, out_hbm.at[pl.ds(pl.multiple_of(row0, 8), L), :])

    return k(logits, idx)


def _combine_body(a_ref, seg_ref, logz_ref, lp_ref):
    B = a_ref.shape[0]
    c = lax.bitwise_and(a_ref[...], 127)
    col = lax.broadcasted_iota(jnp.int32, (B, 128), 1)
    g = jnp.sum(jnp.where(col == c, seg_ref[...], 0.0), axis=-1, keepdims=True)
    lp_ref[...] = g - logz_ref[...]


def _combine(actions, seg, logz):
    B = actions.shape[0]
    return pl.pallas_call(
        _combine_body,
        out_shape=jax.ShapeDtypeStruct((B, 1), jnp.float32),
    )(actions, seg, logz)


def kernel(logits, actions):
    B, V = logits.shape
    a = actions.reshape(B).astype(jnp.int32)
    seg = _sc_gather(logits, a)
    logz, mode = _fused_pass(logits)
    log_probs = _combine(actions.astype(jnp.int32), seg, logz)
    return log_probs, mode


# speedup vs baseline: 1.0416x; 1.0035x over previous
import functools

import jax
import jax.numpy as jnp
from jax import lax
from jax.experimental import pallas as pl
from jax.experimental.pallas import tpu as pltpu
from jax.experimental.pallas import tpu_sc as plsc

_NEG_INF = float("-inf")
_LOG2E = 1.4426950408889634


def _reduce_body(V, C, NBLK, x_ref, logz_ref, mode_ref, m_sc, s_sc, bv_sc, bi_sc):
    j = pl.program_id(0)
    B = x_ref.shape[0]
    col = lax.broadcasted_iota(jnp.int32, (B, C), 1)

    def _stats(masked):
        x = x_ref[...]
        if masked:
            x = jnp.where(j * C + col < V, x, _NEG_INF)
        bmax = jnp.max(x, axis=-1, keepdims=True)
        bidx = jnp.min(jnp.where(x == bmax, col, 2**31 - 1), axis=-1,
                       keepdims=True) + j * C
        bsum = jnp.sum(jnp.exp(x - bmax), axis=-1, keepdims=True)
        return bmax, bidx, bsum

    def _acc(bmax, bidx, bsum):
        m_old = m_sc[...]
        m_new = jnp.maximum(m_old, bmax)
        s_sc[...] = s_sc[...] * jnp.exp(m_old - m_new) + bsum * jnp.exp(bmax - m_new)
        m_sc[...] = m_new
        better = bmax > bv_sc[...]
        bv_sc[...] = jnp.where(better, bmax, bv_sc[...])
        bi_sc[...] = jnp.where(better, bidx, bi_sc[...])

    tail_masked = V % C != 0

    @pl.when(j == 0)
    def _init():
        bmax, bidx, bsum = _stats(masked=tail_masked and NBLK == 1)
        m_sc[...] = bmax
        s_sc[...] = bsum
        bv_sc[...] = bmax
        bi_sc[...] = bidx

    @pl.when((j > 0) & (j < NBLK - 1))
    def _mid():
        _acc(*_stats(masked=False))

    @pl.when((j == NBLK - 1) & (j > 0))
    def _last():
        _acc(*_stats(masked=tail_masked))

    @pl.when(j == NBLK - 1)
    def _fin():
        logz_ref[...] = m_sc[...] + jnp.log(s_sc[...])
        mode_ref[...] = bi_sc[...]


def _fused_pass(logits, C=49152):
    B, V = logits.shape
    NBLK = pl.cdiv(V, C)
    return pl.pallas_call(
        functools.partial(_reduce_body, V, C, NBLK),
        grid=(NBLK,),
        in_specs=[pl.BlockSpec((B, C), lambda j: (0, j))],
        out_specs=[pl.BlockSpec((B, 1), lambda j: (0, 0)),
                   pl.BlockSpec((B, 1), lambda j: (0, 0))],
        out_shape=[jax.ShapeDtypeStruct((B, 1), jnp.float32),
                   jax.ShapeDtypeStruct((B, 1), jnp.int32)],
        scratch_shapes=[pltpu.VMEM((B, 1), jnp.float32),
                        pltpu.VMEM((B, 1), jnp.float32),
                        pltpu.VMEM((B, 1), jnp.float32),
                        pltpu.VMEM((B, 1), jnp.int32)],
    )(logits)


def _sc_gather(logits, idx):
    info = plsc.get_sparse_core_info()
    nc, L = info.num_cores, info.num_lanes
    B, V = logits.shape
    nw = B // L
    mesh = plsc.VectorSubcoreMesh(core_axis_name="c", subcore_axis_name="s")

    @functools.partial(
        pl.kernel, mesh=mesh,
        out_type=jax.ShapeDtypeStruct((B, 128), jnp.float32),
        scratch_types=[pltpu.VMEM((L,), jnp.int32),
                       pltpu.VMEM((L, 8, 128), jnp.float32),
                       pltpu.VMEM((L, 128), jnp.float32)],
    )
    def k(x_hbm, idx_hbm, out_hbm, idx_v, buf_v, seg_v):
        wid = lax.axis_index("s") * nc + lax.axis_index("c")

        @pl.when(wid < nw)
        def _():
            row0 = wid * L
            pltpu.sync_copy(idx_hbm.at[pl.ds(row0, L)], idx_v)
            idx_reg = idx_v[...]
            for i in range(L):
                base = pl.multiple_of(lax.bitwise_and(idx_reg[i], -128), 128)
                r0 = pl.multiple_of(row0 + (i & ~7), 8)
                pltpu.sync_copy(x_hbm.at[pl.ds(r0, 8), pl.ds(base, 128)],
                                buf_v.at[i])
                for k16 in range(8):
                    seg_v[i, pl.ds(16 * k16, 16)] = buf_v[i, i & 7, pl.ds(16 * k16, 16)]
            pltpu.sync_copy(
                seg_v, out_hbm.at[pl.ds(pl.multiple_of(row0, 8), L), :])

    return k(logits, idx)


def _combine_body(a_ref, seg_ref, logz_ref, lp_ref):
    B = a_ref.shape[0]
    c = lax.bitwise_and(a_ref[...], 127)
    col = lax.broadcasted_iota(jnp.int32, (B, 128), 1)
    g = jnp.sum(jnp.where(col == c, seg_ref[...], 0.0), axis=-1, keepdims=True)
    lp_ref[...] = g - logz_ref[...]


def _combine(actions, seg, logz):
    B = actions.shape[0]
    return pl.pallas_call(
        _combine_body,
        out_shape=jax.ShapeDtypeStruct((B, 1), jnp.float32),
    )(actions, seg, logz)


def kernel(logits, actions):
    B, V = logits.shape
    a = actions.reshape(B).astype(jnp.int32)
    seg = _sc_gather(logits, a)
    logz, mode = _fused_pass(logits)
    log_probs = _combine(actions.astype(jnp.int32), seg, logz)
    return log_probs, mode


# speedup vs baseline: 1.0798x; 1.0367x over previous
import functools

import jax
import jax.numpy as jnp
from jax import lax
from jax.experimental import pallas as pl
from jax.experimental.pallas import tpu as pltpu
from jax.experimental.pallas import tpu_sc as plsc

_NEG_INF = float("-inf")
_LOG2E = 1.4426950408889634


def _reduce_body(V, C, NG, K1, NSKIP, x_ref, m_ref, s_ref, bi_ref,
                 m_sc, s_sc, bi_sc):
    j = pl.program_id(0)
    jj = jnp.where(j >= K1, j + NSKIP, j)
    B = x_ref.shape[0]
    col = lax.broadcasted_iota(jnp.int32, (B, C), 1)

    def _stats(masked):
        x = x_ref[...]
        if masked:
            x = jnp.where(jj * C + col < V, x, _NEG_INF)
        bmax = jnp.max(x, axis=-1, keepdims=True)
        bidx = jnp.min(jnp.where(x == bmax, col, 2**31 - 1), axis=-1,
                       keepdims=True) + jj * C
        bsum = jnp.sum(jnp.exp(x - bmax), axis=-1, keepdims=True)
        return bmax, bidx, bsum

    def _acc(bmax, bidx, bsum):
        m_old = m_sc[...]
        m_new = jnp.maximum(m_old, bmax)
        s_sc[...] = s_sc[...] * jnp.exp(m_old - m_new) + bsum * jnp.exp(bmax - m_new)
        m_sc[...] = m_new
        better = bmax > m_old
        bi_sc[...] = jnp.where(better, bidx, bi_sc[...])

    @pl.when(j == 0)
    def _init():
        bmax, bidx, bsum = _stats(masked=NG == 1)
        m_sc[...] = bmax
        s_sc[...] = bsum
        bi_sc[...] = bidx

    @pl.when((j > 0) & (j < NG - 1))
    def _mid():
        _acc(*_stats(masked=False))

    @pl.when((j == NG - 1) & (j > 0))
    def _last():
        _acc(*_stats(masked=True))

    @pl.when(j == NG - 1)
    def _fin():
        m_ref[...] = m_sc[...]
        s_ref[...] = s_sc[...]
        bi_ref[...] = bi_sc[...]


def _fused_pass(logits, C, K1, NSKIP):
    B, V = logits.shape
    NBLK = pl.cdiv(V, C)
    NG = NBLK - NSKIP

    def _in_map(j):
        return (0, jnp.where(j >= K1, j + NSKIP, j))

    return pl.pallas_call(
        functools.partial(_reduce_body, V, C, NG, K1, NSKIP),
        grid=(NG,),
        in_specs=[pl.BlockSpec((B, C), _in_map)],
        out_specs=[pl.BlockSpec((B, 1), lambda j: (0, 0)),
                   pl.BlockSpec((B, 1), lambda j: (0, 0)),
                   pl.BlockSpec((B, 1), lambda j: (0, 0))],
        out_shape=[jax.ShapeDtypeStruct((B, 1), jnp.float32),
                   jax.ShapeDtypeStruct((B, 1), jnp.float32),
                   jax.ShapeDtypeStruct((B, 1), jnp.int32)],
        scratch_shapes=[pltpu.VMEM((B, 1), jnp.float32),
                        pltpu.VMEM((B, 1), jnp.float32),
                        pltpu.VMEM((B, 1), jnp.int32)],
    )(logits)


def _sc_slice_reduce(logits, c0, ncols):
    info = plsc.get_sparse_core_info()
    nc, L = info.num_cores, info.num_lanes
    B, V = logits.shape
    TPG = 4
    cols_pt = ncols // TPG
    q = cols_pt // 1024
    mesh = plsc.VectorSubcoreMesh(core_axis_name="c", subcore_axis_name="s")
    out_sds = (jax.ShapeDtypeStruct((TPG, B, 128), jnp.float32),
               jax.ShapeDtypeStruct((TPG, B, 128), jnp.float32),
               jax.ShapeDtypeStruct((TPG, B, 128), jnp.int32))

    @functools.partial(
        pl.kernel, mesh=mesh, out_type=out_sds,
        scratch_types=[pltpu.VMEM((8, 1024), jnp.float32),
                       pltpu.VMEM((8, 128), jnp.float32),
                       pltpu.VMEM((8, 128), jnp.float32),
                       pltpu.VMEM((8, 128), jnp.int32)],
    )
    def k(x_hbm, pm_hbm, ps_hbm, pi_hbm, buf_v, pm_v, ps_v, pi_v):
        wid = lax.axis_index("s") * nc + lax.axis_index("c")
        g = wid // TPG
        t = lax.rem(wid, TPG)
        r0 = pl.multiple_of(g * 8, 8)
        col0 = c0 + t * cols_pt
        iota = lax.iota(jnp.int32, L)
        init = ([jnp.full((L,), _NEG_INF, jnp.float32)] * 8
                + [jnp.zeros((L,), jnp.float32)] * 8
                + [jnp.zeros((L,), jnp.int32)] * 8)

        def chunk_body(c, carry):
            ms, ss, bis = list(carry[0:8]), list(carry[8:16]), list(carry[16:24])
            off = col0 + c * 1024
            pltpu.sync_copy(
                x_hbm.at[pl.ds(r0, 8), pl.ds(pl.multiple_of(off, 128), 1024)],
                buf_v)
            m_old = list(ms)

            def pass_max(i, ca):
                ms2, bis2 = list(ca[0:8]), list(ca[8:16])
                colv = iota + (off + i * L)
                om, oi = [], []
                for r in range(8):
                    x = buf_v[r, pl.ds(pl.multiple_of(i * L, 8), L)]
                    gt = x > ms2[r]
                    om.append(jnp.where(gt, x, ms2[r]))
                    oi.append(jnp.where(gt, colv, bis2[r]))
                return tuple(om + oi)

            res = lax.fori_loop(0, 1024 // L, pass_max, tuple(ms + bis))
            ms, bis = list(res[0:8]), list(res[8:16])
            ss = [ss[r] * jnp.exp(m_old[r] - ms[r]) for r in range(8)]

            def pass_sum(i, cb):
                out = []
                for r in range(8):
                    x = buf_v[r, pl.ds(pl.multiple_of(i * L, 8), L)]
                    out.append(cb[r] + jnp.exp(x - ms[r]))
                return tuple(out)

            ss = list(lax.fori_loop(0, 1024 // L, pass_sum, tuple(ss)))
            return tuple(ms + ss + bis)

        fin = lax.fori_loop(0, q, chunk_body, tuple(init))
        for r in range(8):
            pm_v[r, pl.ds(0, L)] = fin[r]
            ps_v[r, pl.ds(0, L)] = fin[8 + r]
            pi_v[r, pl.ds(0, L)] = fin[16 + r]
        pltpu.sync_copy(pm_v, pm_hbm.at[t, pl.ds(r0, 8), :])
        pltpu.sync_copy(ps_v, ps_hbm.at[t, pl.ds(r0, 8), :])
        pltpu.sync_copy(pi_v, pi_hbm.at[t, pl.ds(r0, 8), :])

    return k(logits)


def _sc_gather(logits, idx):
    info = plsc.get_sparse_core_info()
    nc, L = info.num_cores, info.num_lanes
    B, V = logits.shape
    nw = B // L
    mesh = plsc.VectorSubcoreMesh(core_axis_name="c", subcore_axis_name="s")

    @functools.partial(
        pl.kernel, mesh=mesh,
        out_type=jax.ShapeDtypeStruct((B, 128), jnp.float32),
        scratch_types=[pltpu.VMEM((L,), jnp.int32),
                       pltpu.VMEM((L, 8, 128), jnp.float32),
                       pltpu.VMEM((L, 128), jnp.float32)],
    )
    def k(x_hbm, idx_hbm, out_hbm, idx_v, buf_v, seg_v):
        wid = lax.axis_index("s") * nc + lax.axis_index("c")

        @pl.when(wid < nw)
        def _():
            row0 = wid * L
            pltpu.sync_copy(idx_hbm.at[pl.ds(row0, L)], idx_v)
            idx_reg = idx_v[...]
            for i in range(L):
                base = pl.multiple_of(lax.bitwise_and(idx_reg[i], -128), 128)
                r0 = pl.multiple_of(row0 + (i & ~7), 8)
                pltpu.sync_copy(x_hbm.at[pl.ds(r0, 8), pl.ds(base, 128)],
                                buf_v.at[i])
                for k16 in range(8):
                    seg_v[i, pl.ds(16 * k16, 16)] = buf_v[i, i & 7, pl.ds(16 * k16, 16)]
            pltpu.sync_copy(
                seg_v, out_hbm.at[pl.ds(pl.multiple_of(row0, 8), L), :])

    return k(logits, idx)


def _merge_body(a_ref, seg_ref, m_ref, s_ref, bi_ref, pm_ref, ps_ref, pi_ref,
                lp_ref, mode_ref):
    B = a_ref.shape[0]
    TPG = pm_ref.shape[0]
    col = lax.broadcasted_iota(jnp.int32, (B, 128), 1)
    valid = col < 16
    m_all = m_ref[...]
    pms = []
    for t in range(TPG):
        pm = jnp.where(valid, pm_ref[t], _NEG_INF)
        pms.append(pm)
        m_all = jnp.maximum(m_all, jnp.max(pm, axis=-1, keepdims=True))
    s_all = s_ref[...] * jnp.exp(m_ref[...] - m_all)
    idx = jnp.where(m_ref[...] == m_all, bi_ref[...], 2**31 - 1)
    for t in range(TPG):
        ps = jnp.where(valid, ps_ref[t], 0.0)
        s_all = s_all + jnp.sum(ps * jnp.exp(pms[t] - m_all), axis=-1,
                                keepdims=True)
        cand = jnp.where(pms[t] == m_all, pi_ref[t], 2**31 - 1)
        idx = jnp.minimum(idx, jnp.min(cand, axis=-1, keepdims=True))
    logz = m_all + jnp.log(s_all)
    c = lax.bitwise_and(a_ref[...], 127)
    g = jnp.sum(jnp.where(col == c, seg_ref[...], 0.0), axis=-1, keepdims=True)
    lp_ref[...] = g - logz
    mode_ref[...] = idx


def _merge(actions, seg, m, s, bi, pm, ps, pi):
    B = actions.shape[0]
    return pl.pallas_call(
        _merge_body,
        out_shape=[jax.ShapeDtypeStruct((B, 1), jnp.float32),
                   jax.ShapeDtypeStruct((B, 1), jnp.int32)],
    )(actions, seg, m, s, bi, pm, ps, pi)


def kernel(logits, actions):
    B, V = logits.shape
    C = 32768
    NBLK = pl.cdiv(V, C)
    NSKIP = 3
    K1 = NBLK - NSKIP - 1
    a = actions.reshape(B).astype(jnp.int32)
    seg = _sc_gather(logits, a)
    pm, ps, pi = _sc_slice_reduce(logits, K1 * C, NSKIP * C)
    m, s, bi = _fused_pass(logits, C, K1, NSKIP)
    log_probs, mode = _merge(actions.astype(jnp.int32), seg, m, s, bi, pm, ps, pi)
    return log_probs, mode


# speedup vs baseline: 1.0983x; 1.0171x over previous
import functools

import jax
import jax.numpy as jnp
from jax import lax
from jax.experimental import pallas as pl
from jax.experimental.pallas import tpu as pltpu
from jax.experimental.pallas import tpu_sc as plsc

_NEG_INF = float("-inf")
_LOG2E = 1.4426950408889634


def _reduce_body(V, C, NG, K1, NSKIP, x_ref, m_ref, s_ref, bi_ref,
                 m_sc, s_sc, bi_sc):
    j = pl.program_id(0)
    jj = jnp.where(j >= K1, j + NSKIP, j)
    B = x_ref.shape[0]
    col = lax.broadcasted_iota(jnp.int32, (B, C), 1)

    def _stats(masked):
        x = x_ref[...]
        if masked:
            x = jnp.where(jj * C + col < V, x, _NEG_INF)
        bmax = jnp.max(x, axis=-1, keepdims=True)
        bidx = jnp.min(jnp.where(x == bmax, col, 2**31 - 1), axis=-1,
                       keepdims=True) + jj * C
        bsum = jnp.sum(jnp.exp(x - bmax), axis=-1, keepdims=True)
        return bmax, bidx, bsum

    def _acc(bmax, bidx, bsum):
        m_old = m_sc[...]
        m_new = jnp.maximum(m_old, bmax)
        s_sc[...] = s_sc[...] * jnp.exp(m_old - m_new) + bsum * jnp.exp(bmax - m_new)
        m_sc[...] = m_new
        better = bmax > m_old
        bi_sc[...] = jnp.where(better, bidx, bi_sc[...])

    @pl.when(j == 0)
    def _init():
        bmax, bidx, bsum = _stats(masked=NG == 1)
        m_sc[...] = bmax
        s_sc[...] = bsum
        bi_sc[...] = bidx

    @pl.when((j > 0) & (j < NG - 1))
    def _mid():
        _acc(*_stats(masked=False))

    @pl.when((j == NG - 1) & (j > 0))
    def _last():
        _acc(*_stats(masked=True))

    @pl.when(j == NG - 1)
    def _fin():
        m_ref[...] = m_sc[...]
        s_ref[...] = s_sc[...]
        bi_ref[...] = bi_sc[...]


def _fused_pass(logits, C, K1, NSKIP):
    B, V = logits.shape
    NBLK = pl.cdiv(V, C)
    NG = NBLK - NSKIP

    def _in_map(j):
        return (0, jnp.where(j >= K1, j + NSKIP, j))

    return pl.pallas_call(
        functools.partial(_reduce_body, V, C, NG, K1, NSKIP),
        grid=(NG,),
        in_specs=[pl.BlockSpec((B, C), _in_map)],
        out_specs=[pl.BlockSpec((B, 1), lambda j: (0, 0)),
                   pl.BlockSpec((B, 1), lambda j: (0, 0)),
                   pl.BlockSpec((B, 1), lambda j: (0, 0))],
        out_shape=[jax.ShapeDtypeStruct((B, 1), jnp.float32),
                   jax.ShapeDtypeStruct((B, 1), jnp.float32),
                   jax.ShapeDtypeStruct((B, 1), jnp.int32)],
        scratch_shapes=[pltpu.VMEM((B, 1), jnp.float32),
                        pltpu.VMEM((B, 1), jnp.float32),
                        pltpu.VMEM((B, 1), jnp.int32)],
    )(logits)


def _sc_slice_reduce(logits, c0, ncols):
    info = plsc.get_sparse_core_info()
    nc, L = info.num_cores, info.num_lanes
    B, V = logits.shape
    TPG = 4
    cols_pt = ncols // TPG
    q = cols_pt // 1024
    mesh = plsc.VectorSubcoreMesh(core_axis_name="c", subcore_axis_name="s")
    out_sds = (jax.ShapeDtypeStruct((TPG, B, 128), jnp.float32),
               jax.ShapeDtypeStruct((TPG, B, 128), jnp.float32),
               jax.ShapeDtypeStruct((TPG, B, 128), jnp.int32))

    @functools.partial(
        pl.kernel, mesh=mesh, out_type=out_sds,
        scratch_types=[pltpu.VMEM((8, 1024), jnp.float32),
                       pltpu.VMEM((8, 128), jnp.float32),
                       pltpu.VMEM((8, 128), jnp.float32),
                       pltpu.VMEM((8, 128), jnp.int32)],
    )
    def k(x_hbm, pm_hbm, ps_hbm, pi_hbm, buf_v, pm_v, ps_v, pi_v):
        wid = lax.axis_index("s") * nc + lax.axis_index("c")
        g = wid // TPG
        t = lax.rem(wid, TPG)
        r0 = pl.multiple_of(g * 8, 8)
        col0 = c0 + t * cols_pt
        iota = lax.iota(jnp.int32, L)
        init = ([jnp.full((L,), _NEG_INF, jnp.float32)] * 8
                + [jnp.zeros((L,), jnp.float32)] * 8
                + [jnp.zeros((L,), jnp.int32)] * 8)

        def chunk_body(c, carry):
            ms, ss, bis = list(carry[0:8]), list(carry[8:16]), list(carry[16:24])
            off = col0 + c * 1024
            pltpu.sync_copy(
                x_hbm.at[pl.ds(r0, 8), pl.ds(pl.multiple_of(off, 128), 1024)],
                buf_v)
            m_old = list(ms)

            def pass_max(i, ca):
                ms2, bis2 = list(ca[0:8]), list(ca[8:16])
                colv = iota + (off + i * L)
                om, oi = [], []
                for r in range(8):
                    x = buf_v[r, pl.ds(pl.multiple_of(i * L, 8), L)]
                    gt = x > ms2[r]
                    om.append(jnp.where(gt, x, ms2[r]))
                    oi.append(jnp.where(gt, colv, bis2[r]))
                return tuple(om + oi)

            res = lax.fori_loop(0, 1024 // L, pass_max, tuple(ms + bis))
            ms, bis = list(res[0:8]), list(res[8:16])
            ss = [ss[r] * jnp.exp(m_old[r] - ms[r]) for r in range(8)]

            def pass_sum(i, cb):
                out = []
                for r in range(8):
                    x = buf_v[r, pl.ds(pl.multiple_of(i * L, 8), L)]
                    out.append(cb[r] + jnp.exp(x - ms[r]))
                return tuple(out)

            ss = list(lax.fori_loop(0, 1024 // L, pass_sum, tuple(ss)))
            return tuple(ms + ss + bis)

        fin = lax.fori_loop(0, q, chunk_body, tuple(init))
        for r in range(8):
            pm_v[r, pl.ds(0, L)] = fin[r]
            ps_v[r, pl.ds(0, L)] = fin[8 + r]
            pi_v[r, pl.ds(0, L)] = fin[16 + r]
        pltpu.sync_copy(pm_v, pm_hbm.at[t, pl.ds(r0, 8), :])
        pltpu.sync_copy(ps_v, ps_hbm.at[t, pl.ds(r0, 8), :])
        pltpu.sync_copy(pi_v, pi_hbm.at[t, pl.ds(r0, 8), :])

    return k(logits)


def _sc_gather(logits, idx):
    info = plsc.get_sparse_core_info()
    nc, L = info.num_cores, info.num_lanes
    B, V = logits.shape
    nw = B // L
    mesh = plsc.VectorSubcoreMesh(core_axis_name="c", subcore_axis_name="s")

    @functools.partial(
        pl.kernel, mesh=mesh,
        out_type=jax.ShapeDtypeStruct((B, 128), jnp.float32),
        scratch_types=[pltpu.VMEM((L,), jnp.int32),
                       pltpu.VMEM((L, 8, 128), jnp.float32),
                       pltpu.VMEM((L, 128), jnp.float32)],
    )
    def k(x_hbm, idx_hbm, out_hbm, idx_v, buf_v, seg_v):
        wid = lax.axis_index("s") * nc + lax.axis_index("c")

        @pl.when(wid < nw)
        def _():
            row0 = wid * L
            pltpu.sync_copy(idx_hbm.at[pl.ds(row0, L)], idx_v)
            idx_reg = idx_v[...]
            for i in range(L):
                base = pl.multiple_of(lax.bitwise_and(idx_reg[i], -128), 128)
                r0 = pl.multiple_of(row0 + (i & ~7), 8)
                pltpu.sync_copy(x_hbm.at[pl.ds(r0, 8), pl.ds(base, 128)],
                                buf_v.at[i])
                for k16 in range(8):
                    seg_v[i, pl.ds(16 * k16, 16)] = buf_v[i, i & 7, pl.ds(16 * k16, 16)]
            pltpu.sync_copy(
                seg_v, out_hbm.at[pl.ds(pl.multiple_of(row0, 8), L), :])

    return k(logits, idx)


def _merge_body(a_ref, seg_ref, m_ref, s_ref, bi_ref, pm_ref, ps_ref, pi_ref,
                lp_ref, mode_ref):
    B = a_ref.shape[0]
    TPG = pm_ref.shape[0]
    col = lax.broadcasted_iota(jnp.int32, (B, 128), 1)
    valid = col < 16
    m_all = m_ref[...]
    pms = []
    for t in range(TPG):
        pm = jnp.where(valid, pm_ref[t], _NEG_INF)
        pms.append(pm)
        m_all = jnp.maximum(m_all, jnp.max(pm, axis=-1, keepdims=True))
    s_all = s_ref[...] * jnp.exp(m_ref[...] - m_all)
    idx = jnp.where(m_ref[...] == m_all, bi_ref[...], 2**31 - 1)
    for t in range(TPG):
        ps = jnp.where(valid, ps_ref[t], 0.0)
        s_all = s_all + jnp.sum(ps * jnp.exp(pms[t] - m_all), axis=-1,
                                keepdims=True)
        cand = jnp.where(pms[t] == m_all, pi_ref[t], 2**31 - 1)
        idx = jnp.minimum(idx, jnp.min(cand, axis=-1, keepdims=True))
    logz = m_all + jnp.log(s_all)
    c = lax.bitwise_and(a_ref[...], 127)
    g = jnp.sum(jnp.where(col == c, seg_ref[...], 0.0), axis=-1, keepdims=True)
    lp_ref[...] = g - logz
    mode_ref[...] = idx


def _merge(actions, seg, m, s, bi, pm, ps, pi):
    B = actions.shape[0]
    return pl.pallas_call(
        _merge_body,
        out_shape=[jax.ShapeDtypeStruct((B, 1), jnp.float32),
                   jax.ShapeDtypeStruct((B, 1), jnp.int32)],
    )(actions, seg, m, s, bi, pm, ps, pi)


def kernel(logits, actions):
    B, V = logits.shape
    C = 32768
    NBLK = pl.cdiv(V, C)
    NSKIP = 4
    K1 = NBLK - NSKIP - 1
    a = actions.reshape(B).astype(jnp.int32)
    seg = _sc_gather(logits, a)
    pm, ps, pi = _sc_slice_reduce(logits, K1 * C, NSKIP * C)
    m, s, bi = _fused_pass(logits, C, K1, NSKIP)
    log_probs, mode = _merge(actions.astype(jnp.int32), seg, m, s, bi, pm, ps, pi)
    return log_probs, mode
